# Initial kernel scaffold; baseline (speedup 1.0000x reference)
import jax
import jax.numpy as jnp
from jax.experimental import pallas as pl


def kernel(nq, ns, nv, nt, left_indices, right_indices, rs_input, q_coeff, lc_coeff, lcuts_W):
    raise NotImplementedError("write your pallas kernel here")


# speedup vs baseline: 3.6608x; 3.6608x over previous
import functools

import jax
import jax.numpy as jnp
from jax import lax
from jax.experimental import pallas as pl
from jax.experimental.pallas import tpu as pltpu
from jax.experimental.pallas import tpu_sc as plsc

NC = 2
NS = 16
L = 16
CHUNK = 128


def _sc_body(ns_hbm, lidx_hbm, ridx_hbm, rs_hbm, lc_hbm, w_hbm, zeros_hbm,
             partial_hbm,
             acc, lidx_v, ridx_v, rsbuf, cutbuf, rows_l, rows_r, lc_v, w_v,
             sem_l, sem_r, *, n_pad, d_feat, edges_per_tile, zrows):
    cid = lax.axis_index("c")
    sid = lax.axis_index("s")
    wid = cid * NS + sid
    fgroups = d_feat // L
    rows_per_tile = n_pad // NS
    n_chunks = edges_per_tile // CHUNK

    pltpu.sync_copy(lc_hbm, lc_v)
    pltpu.sync_copy(w_hbm, w_v)
    zero16 = jnp.zeros((L,), jnp.int32)

    row0 = sid * rows_per_tile
    pltpu.sync_copy(zeros_hbm, acc.at[pl.ds(row0, rows_per_tile)])
    plsc.subcore_barrier()

    def chunk_body(c, carry):
        lc_splat, w_splat = carry
        base = wid * edges_per_tile + c * CHUNK
        pltpu.sync_copy(lidx_hbm.at[pl.ds(base, CHUNK)], lidx_v)
        pltpu.sync_copy(ridx_hbm.at[pl.ds(base, CHUNK)], ridx_v)
        pltpu.sync_copy(rs_hbm.at[pl.ds(base, CHUNK)], rsbuf)
        cp_l = pltpu.async_copy(ns_hbm.at[lidx_v], rows_l, sem_l)
        cp_r = pltpu.async_copy(ns_hbm.at[ridx_v], rows_r, sem_r)
        for g in range(CHUNK // L):
            sl = pl.ds(g * L, L)
            x = rsbuf[sl] - lc_splat
            u = x * w_splat
            neg = x < 0.0
            safe = jnp.where(neg, u, 1.0)
            cutbuf[sl] = jnp.where(neg, jnp.exp(-1.0 / safe), 0.0)
        cp_l.wait()
        cp_r.wait()

        def edge_body(i, _):
            csplat = plsc.load_gather(cutbuf, [zero16 + i])
            for f in range(fgroups):
                sl = pl.ds(f * L, L)
                rows_l[i, sl] = (rows_l[i, sl] + rows_r[i, sl]) * csplat
            return 0
        lax.fori_loop(0, CHUNK, edge_body, 0)

        pltpu.sync_copy(rows_l, acc.at[lidx_v], add=True)
        pltpu.sync_copy(rows_l, acc.at[ridx_v], add=True)
        return carry
    lax.fori_loop(0, n_chunks, chunk_body, (lc_v[...], w_v[...]))
    plsc.subcore_barrier()

    pltpu.sync_copy(acc.at[pl.ds(row0, rows_per_tile)],
                    partial_hbm.at[pl.ds(cid * n_pad + row0, rows_per_tile)])


def _combine_body(a_ref, b_ref, o_ref):
    o_ref[...] = a_ref[...] + b_ref[...]


@functools.partial(jax.jit, static_argnames=())
def _twobody_impl(ns, l_p, r_p, rs_p, lc16, w16):
    n_nodes, d_feat = ns.shape
    e_pad = l_p.shape[0]
    edges_per_tile = e_pad // (NC * NS)
    n_pad = ((n_nodes + 8 * NS - 1) // (8 * NS)) * (8 * NS)
    zrows = 128

    mesh = plsc.VectorSubcoreMesh(core_axis_name="c", subcore_axis_name="s")
    body = functools.partial(_sc_body, n_pad=n_pad, d_feat=d_feat,
                             edges_per_tile=edges_per_tile, zrows=zrows)
    sc_call = pl.kernel(
        body,
        out_type=jax.ShapeDtypeStruct((NC * n_pad, d_feat), jnp.float32),
        mesh=mesh,
        compiler_params=pltpu.CompilerParams(needs_layout_passes=False),
        scratch_types=[
            pltpu.VMEM_SHARED((n_pad, d_feat), jnp.float32),
            pltpu.VMEM((CHUNK,), jnp.int32),
            pltpu.VMEM((CHUNK,), jnp.int32),
            pltpu.VMEM((CHUNK,), jnp.float32),
            pltpu.VMEM((CHUNK,), jnp.float32),
            pltpu.VMEM((CHUNK, d_feat), jnp.float32),
            pltpu.VMEM((CHUNK, d_feat), jnp.float32),
            pltpu.VMEM((L,), jnp.float32),
            pltpu.VMEM((L,), jnp.float32),
            pltpu.SemaphoreType.DMA,
            pltpu.SemaphoreType.DMA,
        ],
    )
    zeros_src = jnp.zeros((n_pad // NS, d_feat), jnp.float32)
    partial = sc_call(ns, l_p, r_p, rs_p, lc16, w16, zeros_src)

    rows_blk = 2000
    n_blocks = n_nodes // rows_blk
    combine = pl.pallas_call(
        _combine_body,
        grid=(n_blocks,),
        in_specs=[
            pl.BlockSpec((rows_blk, d_feat), lambda i: (i, 0)),
            pl.BlockSpec((rows_blk, d_feat), lambda i: (i, 0)),
        ],
        out_specs=pl.BlockSpec((rows_blk, d_feat), lambda i: (i, 0)),
        out_shape=jax.ShapeDtypeStruct((n_nodes, d_feat), jnp.float32),
    )
    return combine(partial[:n_nodes], partial[n_pad:n_pad + n_nodes])


def kernel(nq, ns, nv, nt, left_indices, right_indices, rs_input, q_coeff,
           lc_coeff, lcuts_W):
    n_nodes = ns.shape[0]
    e = left_indices.shape[0]
    tile_chunk = NC * NS * CHUNK
    e_pad = ((e + tile_chunk - 1) // tile_chunk) * tile_chunk
    pad = e_pad - e
    pad_idx = jnp.arange(pad, dtype=jnp.int32) % n_nodes
    l_p = jnp.concatenate([left_indices.astype(jnp.int32), pad_idx])
    r_p = jnp.concatenate([right_indices.astype(jnp.int32), pad_idx])
    rs_p = jnp.concatenate([rs_input[:, 0],
                            jnp.full((pad,), jnp.inf, jnp.float32)])
    lc16 = jnp.broadcast_to(lc_coeff.reshape(()), (L,)).astype(jnp.float32)
    w16 = jnp.broadcast_to(lcuts_W.reshape(()), (L,)).astype(jnp.float32)
    return _twobody_impl(ns, l_p, r_p, rs_p, lc16, w16)


# speedup vs baseline: 3.8351x; 1.0476x over previous
import functools

import jax
import jax.numpy as jnp
from jax import lax
from jax.experimental import pallas as pl
from jax.experimental.pallas import tpu as pltpu
from jax.experimental.pallas import tpu_sc as plsc

NC = 2
NS = 16
L = 16
CHUNK = 64


def _sc_body(ns_hbm, lidx_hbm, ridx_hbm, rs_hbm, lc_hbm, w_hbm, zeros_hbm,
             partial_hbm,
             acc, lidx0, ridx0, rs0, rows_l0, rows_r0,
             lidx1, ridx1, rs1, rows_l1, rows_r1, cutbuf, lc_v, w_v,
             sem_l0, sem_r0, sem_l1, sem_r1,
             *, n_pad, d_feat, edges_per_tile, zrows):
    cid = lax.axis_index("c")
    sid = lax.axis_index("s")
    wid = cid * NS + sid
    fgroups = d_feat // L
    rows_per_tile = n_pad // NS
    n_chunks = edges_per_tile // CHUNK
    last = n_chunks - 1

    pltpu.sync_copy(lc_hbm, lc_v)
    pltpu.sync_copy(w_hbm, w_v)
    zero16 = jnp.zeros((L,), jnp.int32)

    row0 = sid * rows_per_tile
    pltpu.sync_copy(zeros_hbm, acc.at[pl.ds(row0, rows_per_tile)])
    plsc.subcore_barrier()

    tbase = wid * edges_per_tile
    bufs = ((lidx0, ridx0, rs0, rows_l0, rows_r0, sem_l0, sem_r0),
            (lidx1, ridx1, rs1, rows_l1, rows_r1, sem_l1, sem_r1))

    def issue(c, b):
        lidx_v, ridx_v, rsbuf, rows_l, rows_r, sem_l, sem_r = bufs[b]
        base = tbase + c * CHUNK
        pltpu.sync_copy(lidx_hbm.at[pl.ds(base, CHUNK)], lidx_v)
        pltpu.sync_copy(ridx_hbm.at[pl.ds(base, CHUNK)], ridx_v)
        pltpu.sync_copy(rs_hbm.at[pl.ds(base, CHUNK)], rsbuf)
        cp_l = pltpu.async_copy(ns_hbm.at[lidx_v], rows_l, sem_l)
        cp_r = pltpu.async_copy(ns_hbm.at[ridx_v], rows_r, sem_r)
        return cp_l, cp_r

    def consume(b, lc_splat, w_splat):
        lidx_v, ridx_v, rsbuf, rows_l, rows_r, sem_l, sem_r = bufs[b]
        pltpu.make_async_copy(ns_hbm.at[lidx_v], rows_l, sem_l).wait()
        pltpu.make_async_copy(ns_hbm.at[ridx_v], rows_r, sem_r).wait()
        for g in range(CHUNK // L):
            sl = pl.ds(g * L, L)
            x = rsbuf[sl] - lc_splat
            u = x * w_splat
            neg = x < 0.0
            safe = jnp.where(neg, u, 1.0)
            cutbuf[pl.ds(g * L, L)] = jnp.where(neg, jnp.exp(-1.0 / safe), 0.0)

        def edge_body(i2, _):
            for j in range(2):
                i = i2 * 2 + j
                csplat = plsc.load_gather(cutbuf, [zero16 + i])
                for f in range(fgroups):
                    sl = pl.ds(f * L, L)
                    rows_l[i, sl] = (rows_l[i, sl] + rows_r[i, sl]) * csplat
            return 0
        lax.fori_loop(0, CHUNK // 2, edge_body, 0)
        pltpu.sync_copy(rows_l, acc.at[lidx_v], add=True)
        pltpu.sync_copy(rows_l, acc.at[ridx_v], add=True)

    def outer(k, carry):
        lc_splat, w_splat = carry
        c0 = k * 2
        issue(jnp.minimum(c0 + 1, last), 1)
        consume(0, lc_splat, w_splat)
        issue(jnp.minimum(c0 + 2, last), 0)
        consume(1, lc_splat, w_splat)
        return carry

    issue(0, 0)
    lax.fori_loop(0, n_chunks // 2, outer, (lc_v[...], w_v[...]))
    pltpu.make_async_copy(ns_hbm.at[lidx0], rows_l0, sem_l0).wait()
    pltpu.make_async_copy(ns_hbm.at[ridx0], rows_r0, sem_r0).wait()
    plsc.subcore_barrier()

    pltpu.sync_copy(acc.at[pl.ds(row0, rows_per_tile)],
                    partial_hbm.at[pl.ds(cid * n_pad + row0, rows_per_tile)])


def _combine_body(a_ref, b_ref, o_ref):
    o_ref[...] = a_ref[...] + b_ref[...]


@functools.partial(jax.jit, static_argnames=())
def _twobody_impl(ns, l_p, r_p, rs_p, lc16, w16):
    n_nodes, d_feat = ns.shape
    e_pad = l_p.shape[0]
    edges_per_tile = e_pad // (NC * NS)
    n_pad = ((n_nodes + 8 * NS - 1) // (8 * NS)) * (8 * NS)
    zrows = 128

    mesh = plsc.VectorSubcoreMesh(core_axis_name="c", subcore_axis_name="s")
    body = functools.partial(_sc_body, n_pad=n_pad, d_feat=d_feat,
                             edges_per_tile=edges_per_tile, zrows=zrows)
    sc_call = pl.kernel(
        body,
        out_type=jax.ShapeDtypeStruct((NC * n_pad, d_feat), jnp.float32),
        mesh=mesh,
        compiler_params=pltpu.CompilerParams(needs_layout_passes=False),
        scratch_types=[
            pltpu.VMEM_SHARED((n_pad, d_feat), jnp.float32),
            pltpu.VMEM((CHUNK,), jnp.int32),
            pltpu.VMEM((CHUNK,), jnp.int32),
            pltpu.VMEM((CHUNK,), jnp.float32),
            pltpu.VMEM((CHUNK, d_feat), jnp.float32),
            pltpu.VMEM((CHUNK, d_feat), jnp.float32),
            pltpu.VMEM((CHUNK,), jnp.int32),
            pltpu.VMEM((CHUNK,), jnp.int32),
            pltpu.VMEM((CHUNK,), jnp.float32),
            pltpu.VMEM((CHUNK, d_feat), jnp.float32),
            pltpu.VMEM((CHUNK, d_feat), jnp.float32),
            pltpu.VMEM((CHUNK,), jnp.float32),
            pltpu.VMEM((L,), jnp.float32),
            pltpu.VMEM((L,), jnp.float32),
            pltpu.SemaphoreType.DMA,
            pltpu.SemaphoreType.DMA,
            pltpu.SemaphoreType.DMA,
            pltpu.SemaphoreType.DMA,
        ],
    )
    zeros_src = jnp.zeros((n_pad // NS, d_feat), jnp.float32)
    partial = sc_call(ns, l_p, r_p, rs_p, lc16, w16, zeros_src)

    rows_blk = 2000
    n_blocks = n_nodes // rows_blk
    combine = pl.pallas_call(
        _combine_body,
        grid=(n_blocks,),
        in_specs=[
            pl.BlockSpec((rows_blk, d_feat), lambda i: (i, 0)),
            pl.BlockSpec((rows_blk, d_feat), lambda i: (i, 0)),
        ],
        out_specs=pl.BlockSpec((rows_blk, d_feat), lambda i: (i, 0)),
        out_shape=jax.ShapeDtypeStruct((n_nodes, d_feat), jnp.float32),
    )
    return combine(partial[:n_nodes], partial[n_pad:n_pad + n_nodes])


def kernel(nq, ns, nv, nt, left_indices, right_indices, rs_input, q_coeff,
           lc_coeff, lcuts_W):
    n_nodes = ns.shape[0]
    e = left_indices.shape[0]
    tile_chunk = NC * NS * CHUNK * 2
    e_pad = ((e + tile_chunk - 1) // tile_chunk) * tile_chunk
    pad = e_pad - e
    pad_idx = jnp.arange(pad, dtype=jnp.int32) % n_nodes
    l_p = jnp.concatenate([left_indices.astype(jnp.int32), pad_idx])
    r_p = jnp.concatenate([right_indices.astype(jnp.int32), pad_idx])
    rs_p = jnp.concatenate([rs_input[:, 0],
                            jnp.full((pad,), jnp.inf, jnp.float32)])
    lc16 = jnp.broadcast_to(lc_coeff.reshape(()), (L,)).astype(jnp.float32)
    w16 = jnp.broadcast_to(lcuts_W.reshape(()), (L,)).astype(jnp.float32)
    return _twobody_impl(ns, l_p, r_p, rs_p, lc16, w16)


# speedup vs baseline: 4.5084x; 1.1756x over previous
import functools

import jax
import jax.numpy as jnp
from jax import lax
from jax.experimental import pallas as pl
from jax.experimental.pallas import tpu as pltpu
from jax.experimental.pallas import tpu_sc as plsc

NC = 2
NS = 16
L = 16
CHUNK = 64


def _sc_body(ns_hbm, lidx_hbm, ridx_hbm, rs_hbm, lc_hbm, w_hbm, zeros_hbm,
             partial_hbm,
             acc, lidx0, ridx0, rs0, rows_l0, rows_r0,
             lidx1, ridx1, rs1, rows_l1, rows_r1, cutbuf, lc_v, w_v,
             sem_l0, sem_r0, sem_l1, sem_r1, sem_i0, sem_i1,
             *, n_pad, d_feat, edges_per_tile, zrows):
    cid = lax.axis_index("c")
    sid = lax.axis_index("s")
    wid = cid * NS + sid
    fgroups = d_feat // L
    rows_per_tile = n_pad // NS
    n_chunks = edges_per_tile // CHUNK
    last = n_chunks - 1

    pltpu.sync_copy(lc_hbm, lc_v)
    pltpu.sync_copy(w_hbm, w_v)
    zero16 = jnp.zeros((L,), jnp.int32)

    row0 = sid * rows_per_tile
    pltpu.sync_copy(zeros_hbm, acc.at[pl.ds(row0, rows_per_tile)])
    plsc.subcore_barrier()

    tbase = wid * edges_per_tile
    bufs = ((lidx0, ridx0, rs0, rows_l0, rows_r0, sem_l0, sem_r0, sem_i0),
            (lidx1, ridx1, rs1, rows_l1, rows_r1, sem_l1, sem_r1, sem_i1))

    def issue(c, b):
        lidx_v, ridx_v, rsbuf, rows_l, rows_r, sem_l, sem_r, sem_i = bufs[b]
        base = tbase + c * CHUNK
        c1 = pltpu.async_copy(lidx_hbm.at[pl.ds(base, CHUNK)], lidx_v, sem_i)
        c2 = pltpu.async_copy(ridx_hbm.at[pl.ds(base, CHUNK)], ridx_v, sem_i)
        c3 = pltpu.async_copy(rs_hbm.at[pl.ds(base, CHUNK)], rsbuf, sem_i)
        c1.wait()
        c2.wait()
        c3.wait()
        cp_l = pltpu.async_copy(ns_hbm.at[lidx_v], rows_l, sem_l)
        cp_r = pltpu.async_copy(ns_hbm.at[ridx_v], rows_r, sem_r)
        return cp_l, cp_r

    def consume(b, lc_splat, w_splat):
        lidx_v, ridx_v, rsbuf, rows_l, rows_r, sem_l, sem_r, sem_i = bufs[b]
        pltpu.make_async_copy(ns_hbm.at[lidx_v], rows_l, sem_l).wait()
        pltpu.make_async_copy(ns_hbm.at[ridx_v], rows_r, sem_r).wait()
        for g in range(CHUNK // L):
            sl = pl.ds(g * L, L)
            x = rsbuf[sl] - lc_splat
            u = x * w_splat
            neg = x < 0.0
            safe = jnp.where(neg, u, 1.0)
            cutbuf[pl.ds(g * L, L)] = jnp.where(neg, jnp.exp(-1.0 / safe), 0.0)

        def edge_body(i2, _):
            for j in range(2):
                i = i2 * 2 + j
                csplat = plsc.load_gather(cutbuf, [zero16 + i])
                for f in range(fgroups):
                    sl = pl.ds(f * L, L)
                    rows_l[i, sl] = (rows_l[i, sl] + rows_r[i, sl]) * csplat
            return 0
        lax.fori_loop(0, CHUNK // 2, edge_body, 0)
        pltpu.sync_copy(rows_l, acc.at[lidx_v], add=True)
        pltpu.sync_copy(rows_l, acc.at[ridx_v], add=True)

    def outer(k, carry):
        lc_splat, w_splat = carry
        c0 = k * 2
        issue(jnp.minimum(c0 + 1, last), 1)
        consume(0, lc_splat, w_splat)
        issue(jnp.minimum(c0 + 2, last), 0)
        consume(1, lc_splat, w_splat)
        return carry

    issue(0, 0)
    lax.fori_loop(0, n_chunks // 2, outer, (lc_v[...], w_v[...]))
    pltpu.make_async_copy(ns_hbm.at[lidx0], rows_l0, sem_l0).wait()
    pltpu.make_async_copy(ns_hbm.at[ridx0], rows_r0, sem_r0).wait()
    plsc.subcore_barrier()

    pltpu.sync_copy(acc.at[pl.ds(row0, rows_per_tile)],
                    partial_hbm.at[pl.ds(cid * n_pad + row0, rows_per_tile)])


def _combine_body(a_ref, b_ref, o_ref):
    o_ref[...] = a_ref[...] + b_ref[...]


@functools.partial(jax.jit, static_argnames=())
def _twobody_impl(ns, l_p, r_p, rs_p, lc16, w16):
    n_nodes, d_feat = ns.shape
    e_pad = l_p.shape[0]
    edges_per_tile = e_pad // (NC * NS)
    n_pad = ((n_nodes + 8 * NS - 1) // (8 * NS)) * (8 * NS)
    zrows = 128

    mesh = plsc.VectorSubcoreMesh(core_axis_name="c", subcore_axis_name="s")
    body = functools.partial(_sc_body, n_pad=n_pad, d_feat=d_feat,
                             edges_per_tile=edges_per_tile, zrows=zrows)
    sc_call = pl.kernel(
        body,
        out_type=jax.ShapeDtypeStruct((NC * n_pad, d_feat), jnp.float32),
        mesh=mesh,
        compiler_params=pltpu.CompilerParams(needs_layout_passes=False),
        scratch_types=[
            pltpu.VMEM_SHARED((n_pad, d_feat), jnp.float32),
            pltpu.VMEM((CHUNK,), jnp.int32),
            pltpu.VMEM((CHUNK,), jnp.int32),
            pltpu.VMEM((CHUNK,), jnp.float32),
            pltpu.VMEM((CHUNK, d_feat), jnp.float32),
            pltpu.VMEM((CHUNK, d_feat), jnp.float32),
            pltpu.VMEM((CHUNK,), jnp.int32),
            pltpu.VMEM((CHUNK,), jnp.int32),
            pltpu.VMEM((CHUNK,), jnp.float32),
            pltpu.VMEM((CHUNK, d_feat), jnp.float32),
            pltpu.VMEM((CHUNK, d_feat), jnp.float32),
            pltpu.VMEM((CHUNK,), jnp.float32),
            pltpu.VMEM((L,), jnp.float32),
            pltpu.VMEM((L,), jnp.float32),
            pltpu.SemaphoreType.DMA,
            pltpu.SemaphoreType.DMA,
            pltpu.SemaphoreType.DMA,
            pltpu.SemaphoreType.DMA,
            pltpu.SemaphoreType.DMA,
            pltpu.SemaphoreType.DMA,
        ],
    )
    zeros_src = jnp.zeros((n_pad // NS, d_feat), jnp.float32)
    partial = sc_call(ns, l_p, r_p, rs_p, lc16, w16, zeros_src)

    rows_blk = 2000
    n_blocks = n_nodes // rows_blk
    combine = pl.pallas_call(
        _combine_body,
        grid=(n_blocks,),
        in_specs=[
            pl.BlockSpec((rows_blk, d_feat), lambda i: (i, 0)),
            pl.BlockSpec((rows_blk, d_feat), lambda i: (i, 0)),
        ],
        out_specs=pl.BlockSpec((rows_blk, d_feat), lambda i: (i, 0)),
        out_shape=jax.ShapeDtypeStruct((n_nodes, d_feat), jnp.float32),
    )
    return combine(partial[:n_nodes], partial[n_pad:n_pad + n_nodes])


def kernel(nq, ns, nv, nt, left_indices, right_indices, rs_input, q_coeff,
           lc_coeff, lcuts_W):
    n_nodes = ns.shape[0]
    e = left_indices.shape[0]
    tile_chunk = NC * NS * CHUNK * 2
    e_pad = ((e + tile_chunk - 1) // tile_chunk) * tile_chunk
    pad = e_pad - e
    pad_idx = jnp.arange(pad, dtype=jnp.int32) % n_nodes
    l_p = jnp.concatenate([left_indices.astype(jnp.int32), pad_idx])
    r_p = jnp.concatenate([right_indices.astype(jnp.int32), pad_idx])
    rs_p = jnp.concatenate([rs_input[:, 0],
                            jnp.full((pad,), jnp.inf, jnp.float32)])
    lc16 = jnp.broadcast_to(lc_coeff.reshape(()), (L,)).astype(jnp.float32)
    w16 = jnp.broadcast_to(lcuts_W.reshape(()), (L,)).astype(jnp.float32)
    return _twobody_impl(ns, l_p, r_p, rs_p, lc16, w16)
